# baseline (speedup 1.0000x reference)
import functools

import jax
import jax.numpy as jnp
from jax import lax
from jax.experimental import pallas as pl
from jax.experimental.pallas import tpu as pltpu
from jax.experimental.pallas import tpu_sc as plsc

F32 = jnp.float32


def _row_block(n, target=2048):
    best = 8
    for r in range(8, target + 1, 8):
        if n % r == 0:
            best = r
    return best


def _wspec():
    return pl.BlockSpec((256, 256), lambda i: (0, 0))


def _bspec():
    return pl.BlockSpec((1, 256), lambda i: (0, 0))


def _ln(o, g, be):
    mu = jnp.mean(o, axis=-1, keepdims=True)
    var = jnp.mean((o - mu) * (o - mu), axis=-1, keepdims=True)
    return (o - mu) * lax.rsqrt(var + 1e-5) * g + be


def _silu(t):
    return t * lax.logistic(t)


def _encode_node_body(x_ref, w1, b1, w2, b2, g, be, o_ref):
    t = _silu(jnp.dot(x_ref[...], w1[...], preferred_element_type=F32) + b1[...])
    o = jnp.dot(t, w2[...], preferred_element_type=F32) + b2[...]
    o_ref[...] = _ln(o, g[...], be[...])


def _encode_node(x, w1, b1, w2, b2, g, be):
    n = x.shape[0]
    r = _row_block(n)
    return pl.pallas_call(
        _encode_node_body,
        grid=(n // r,),
        in_specs=[
            pl.BlockSpec((r, 256), lambda i: (i, 0)),
            _wspec(), _bspec(), _wspec(), _bspec(), _bspec(), _bspec(),
        ],
        out_specs=pl.BlockSpec((r, 256), lambda i: (i, 0)),
        out_shape=jax.ShapeDtypeStruct((n, 256), F32),
        compiler_params=pltpu.CompilerParams(
            dimension_semantics=("parallel",)),
    )(x, w1, b1, w2, b2, g, be)


def _encode_edge_body(a_ref, w1, b1, w2, b2, g, be, o_ref):
    t = _silu(jnp.dot(a_ref[...], w1[...], preferred_element_type=F32) + b1[...])
    o = jnp.dot(t, w2[...], preferred_element_type=F32) + b2[...]
    o_ref[...] = _ln(o, g[...], be[...])


def _encode_edge(a, w1, b1, w2, b2, g, be):
    e, de = a.shape
    r = _row_block(e)
    return pl.pallas_call(
        _encode_edge_body,
        grid=(e // r,),
        in_specs=[
            pl.BlockSpec((r, de), lambda i: (i, 0)),
            pl.BlockSpec((de, 256), lambda i: (0, 0)),
            _bspec(), _wspec(), _bspec(), _bspec(), _bspec(),
        ],
        out_specs=pl.BlockSpec((r, 256), lambda i: (i, 0)),
        out_shape=jax.ShapeDtypeStruct((e, 256), F32),
        compiler_params=pltpu.CompilerParams(
            dimension_semantics=("parallel",)),
    )(a, w1, b1, w2, b2, g, be)


def _dual_project_body(h_ref, ws, wd, os_ref, od_ref):
    h = h_ref[...]
    os_ref[...] = jnp.dot(h, ws[...], preferred_element_type=F32)
    od_ref[...] = jnp.dot(h, wd[...], preferred_element_type=F32)


def _dual_project(h, ws, wd):
    n = h.shape[0]
    r = _row_block(n)
    return pl.pallas_call(
        _dual_project_body,
        grid=(n // r,),
        in_specs=[pl.BlockSpec((r, 256), lambda i: (i, 0)), _wspec(), _wspec()],
        out_specs=[pl.BlockSpec((r, 256), lambda i: (i, 0))] * 2,
        out_shape=[jax.ShapeDtypeStruct((n, 256), F32)] * 2,
        compiler_params=pltpu.CompilerParams(
            dimension_semantics=("parallel",)),
    )(h, ws, wd)


def _edge_update_body(e_ref, gg_ref, w1, b1, w2, b2, g, be, o_ref):
    t = jnp.dot(e_ref[...], w1[...], preferred_element_type=F32)
    t = _silu(t + gg_ref[...] + b1[...])
    o = jnp.dot(t, w2[...], preferred_element_type=F32) + b2[...]
    o_ref[...] = e_ref[...] + _ln(o, g[...], be[...])


def _edge_update(e, gg, w1, b1, w2, b2, g, be):
    n = e.shape[0]
    r = _row_block(n)
    rspec = pl.BlockSpec((r, 256), lambda i: (i, 0))
    return pl.pallas_call(
        _edge_update_body,
        grid=(n // r,),
        in_specs=[
            rspec, rspec,
            _wspec(), _bspec(), _wspec(), _bspec(), _bspec(), _bspec(),
        ],
        out_specs=rspec,
        out_shape=jax.ShapeDtypeStruct((n, 256), F32),
        compiler_params=pltpu.CompilerParams(
            dimension_semantics=("parallel",)),
    )(e, gg, w1, b1, w2, b2, g, be)


def _node_update_body(h_ref, a_ref, w1h, w1a, b1, w2, b2, g, be, o_ref):
    t = (jnp.dot(h_ref[...], w1h[...], preferred_element_type=F32)
         + jnp.dot(a_ref[...], w1a[...], preferred_element_type=F32))
    t = _silu(t + b1[...])
    o = jnp.dot(t, w2[...], preferred_element_type=F32) + b2[...]
    o_ref[...] = h_ref[...] + _ln(o, g[...], be[...])


def _node_update(h, agg, w1h, w1a, b1, w2, b2, g, be):
    n = h.shape[0]
    r = _row_block(n)
    return pl.pallas_call(
        _node_update_body,
        grid=(n // r,),
        in_specs=[
            pl.BlockSpec((r, 256), lambda i: (i, 0)),
            pl.BlockSpec((r, 256), lambda i: (i, 0)),
            _wspec(), _wspec(), _bspec(), _wspec(), _bspec(), _bspec(), _bspec(),
        ],
        out_specs=pl.BlockSpec((r, 256), lambda i: (i, 0)),
        out_shape=jax.ShapeDtypeStruct((n, 256), F32),
        compiler_params=pltpu.CompilerParams(
            dimension_semantics=("parallel",)),
    )(h, agg, w1h, w1a, b1, w2, b2, g, be)


def _decode_body(h_ref, w1, b1, w2, b2, o_ref):
    t = _silu(jnp.dot(h_ref[...], w1[...], preferred_element_type=F32) + b1[...])
    o_ref[...] = jnp.dot(t, w2[...], preferred_element_type=F32) + b2[...]


def _decode(h, w1, b1, w2, b2):
    n = h.shape[0]
    r = _row_block(n)
    return pl.pallas_call(
        _decode_body,
        grid=(n // r,),
        in_specs=[
            pl.BlockSpec((r, 256), lambda i: (i, 0)),
            _wspec(), _bspec(), _wspec(), _bspec(),
        ],
        out_specs=pl.BlockSpec((r, 256), lambda i: (i, 0)),
        out_shape=jax.ShapeDtypeStruct((n, 256), F32),
        compiler_params=pltpu.CompilerParams(
            dimension_semantics=("parallel",)),
    )(h, w1, b1, w2, b2)


_NC = 2
_NS = 16
_GK = 40
_SK = 80
_WB = 80


def _sc_gather(hs, hd, src, dst):
    n_e = src.shape[0]
    d = hs.shape[1]
    per_w = n_e // (_NC * _NS)
    nchunk = per_w // _GK
    mesh = plsc.VectorSubcoreMesh(core_axis_name="c", subcore_axis_name="s",
                                  num_cores=_NC, num_subcores=_NS)

    @functools.partial(
        pl.kernel,
        out_type=jax.ShapeDtypeStruct((n_e, d), F32),
        mesh=mesh,
        scratch_types=[
            [pltpu.VMEM((_GK,), jnp.int32)] * 2,
            [pltpu.VMEM((_GK,), jnp.int32)] * 2,
            [pltpu.VMEM((_GK, d), F32)] * 2,
            [pltpu.VMEM((_GK, d), F32)] * 2,
            [pltpu.SemaphoreType.DMA] * 2,
            [pltpu.SemaphoreType.DMA] * 2,
        ])
    def k(hs_hbm, hd_hbm, src_hbm, dst_hbm, g_hbm, si, di, bs, bd, ss, sd):
        wid = lax.axis_index("s") * _NC + lax.axis_index("c")
        base = wid * per_w

        def start(j, b):
            off = base + j * _GK
            pltpu.sync_copy(src_hbm.at[pl.ds(off, _GK)], si[b])
            pltpu.sync_copy(dst_hbm.at[pl.ds(off, _GK)], di[b])
            pltpu.async_copy(hs_hbm.at[si[b]], bs[b], ss[b])
            pltpu.async_copy(hd_hbm.at[di[b]], bd[b], sd[b])

        def finish(j, b):
            off = base + j * _GK
            pltpu.make_async_copy(hs_hbm.at[si[b]], bs[b], ss[b]).wait()
            pltpu.make_async_copy(hd_hbm.at[di[b]], bd[b], sd[b]).wait()

            @plsc.parallel_loop(0, _GK, unroll=4)
            def addrow(r):
                for jj in range(d // 16):
                    sl = pl.ds(jj * 16, 16)
                    plsc.addupdate(bs[b].at[r, sl], bd[b][r, sl])
            pltpu.sync_copy(bs[b], g_hbm.at[pl.ds(off, _GK)])

        start(0, 0)
        if nchunk > 1:
            start(1, 1)

        def body(g, carry):
            j0 = 2 * g
            j1 = j0 + 1
            finish(j0, 0)

            @pl.when(j0 + 2 < nchunk)
            def _():
                start(j0 + 2, 0)

            @pl.when(j1 < nchunk)
            def _():
                finish(j1, 1)

            @pl.when(j1 + 2 < nchunk)
            def _():
                start(j1 + 2, 1)

            return carry

        lax.fori_loop(0, (nchunk + 1) // 2, body, 0)

    return k(hs, hd, src, dst)


def _sc_segsum(e, dst, n):
    n_e, d = e.shape
    dh = d // 2
    per_tile = n_e // _NS
    nchunk = per_tile // _SK
    npad = _NS * _WB * ((n + _NS * _WB - 1) // (_NS * _WB))
    rows_per_tile = npad // _NS
    nwb = rows_per_tile // _WB
    mesh = plsc.VectorSubcoreMesh(core_axis_name="c", subcore_axis_name="s",
                                  num_cores=_NC, num_subcores=_NS)

    @functools.partial(
        pl.kernel,
        out_type=jax.ShapeDtypeStruct((npad, d), F32),
        mesh=mesh,
        scratch_types=[
            [pltpu.VMEM((_SK,), jnp.int32)] * 2,
            [pltpu.VMEM((_SK, dh), F32)] * 2,
            pltpu.VMEM((_WB, dh), F32),
            pltpu.VMEM_SHARED((npad, dh), F32),
            [pltpu.SemaphoreType.DMA] * 2,
            [pltpu.SemaphoreType.DMA] * 2,
        ])
    def k(e_hbm, dst_hbm, agg_hbm, idxb, ebuf, wbuf, table, six, sro):
        c = lax.axis_index("c")
        s = lax.axis_index("s")
        col0 = c * dh

        zero16 = jnp.zeros((16,), F32)

        def zrow(r, carry):
            for jj in range(dh // 16):
                wbuf[r, pl.ds(jj * 16, 16)] = zero16
            return carry

        lax.fori_loop(0, _WB, zrow, 0)

        def ztab(t, carry):
            pltpu.sync_copy(
                wbuf, table.at[pl.ds(s * rows_per_tile + t * _WB, _WB)])
            return carry

        lax.fori_loop(0, nwb, ztab, 0)
        plsc.subcore_barrier()

        def start(j, b):
            off = s * per_tile + j * _SK
            pltpu.async_copy(dst_hbm.at[pl.ds(off, _SK)], idxb[b], six[b])
            pltpu.async_copy(e_hbm.at[pl.ds(off, _SK), pl.ds(col0, dh)],
                             ebuf[b], sro[b])

        def finish(j, b):
            off = s * per_tile + j * _SK
            pltpu.make_async_copy(
                dst_hbm.at[pl.ds(off, _SK)], idxb[b], six[b]).wait()
            pltpu.make_async_copy(
                e_hbm.at[pl.ds(off, _SK), pl.ds(col0, dh)],
                ebuf[b], sro[b]).wait()
            pltpu.sync_copy(ebuf[b], table.at[idxb[b]], add=True)

        start(0, 0)
        if nchunk > 1:
            start(1, 1)

        def body(g, carry):
            j0 = 2 * g
            j1 = j0 + 1
            finish(j0, 0)

            @pl.when(j0 + 2 < nchunk)
            def _():
                start(j0 + 2, 0)

            @pl.when(j1 < nchunk)
            def _():
                finish(j1, 1)

            @pl.when(j1 + 2 < nchunk)
            def _():
                start(j1 + 2, 1)

            return carry

        lax.fori_loop(0, (nchunk + 1) // 2, body, 0)
        plsc.subcore_barrier()

        def wb(t, carry):
            r0 = s * rows_per_tile + t * _WB
            pltpu.sync_copy(table.at[pl.ds(r0, _WB)], wbuf)
            pltpu.sync_copy(wbuf, agg_hbm.at[pl.ds(r0, _WB),
                                             pl.ds(col0, dh)])
            return carry

        lax.fori_loop(0, nwb, wb, 0)

    return k(e, dst)


def kernel(x, edge_index, edge_attr,
           ne_W1, ne_b1, ne_W2, ne_b2, ne_g, ne_be,
           ee_W1, ee_b1, ee_W2, ee_b2, ee_g, ee_be,
           pe_W1, pe_b1, pe_W2, pe_b2, pe_g, pe_be,
           pn_W1, pn_b1, pn_W2, pn_b2, pn_g, pn_be,
           de_W1, de_b1, de_W2, de_b2):
    n, d = x.shape
    num_layers = pe_W1.shape[0]
    src = edge_index[0]
    dst = edge_index[1]

    r1 = lambda b: b.reshape(1, -1)

    h = _encode_node(x, ne_W1, r1(ne_b1), ne_W2, r1(ne_b2), r1(ne_g), r1(ne_be))
    e = _encode_edge(edge_attr, ee_W1, r1(ee_b1), ee_W2, r1(ee_b2),
                     r1(ee_g), r1(ee_be))

    for i in range(num_layers):
        w1 = pe_W1[i]
        hs, hd = _dual_project(h, w1[d:2 * d], w1[2 * d:])
        gg = _sc_gather(hs, hd, src, dst)
        e = _edge_update(e, gg, w1[:d], r1(pe_b1[i]), pe_W2[i],
                         r1(pe_b2[i]), r1(pe_g[i]), r1(pe_be[i]))
        agg = _sc_segsum(e, dst, n)
        h = _node_update(h, agg, pn_W1[i][:d], pn_W1[i][d:], r1(pn_b1[i]),
                         pn_W2[i], r1(pn_b2[i]), r1(pn_g[i]), r1(pn_be[i]))

    return _decode(h, de_W1, r1(de_b1), de_W2, r1(de_b2))


# speedup vs baseline: 1.2254x; 1.2254x over previous
import functools

import jax
import jax.numpy as jnp
from jax import lax
from jax.experimental import pallas as pl
from jax.experimental.pallas import tpu as pltpu
from jax.experimental.pallas import tpu_sc as plsc

F32 = jnp.float32


def _row_block(n, target=2048):
    best = 8
    for r in range(8, target + 1, 8):
        if n % r == 0:
            best = r
    return best


def _wspec():
    return pl.BlockSpec((256, 256), lambda i: (0, 0))


def _bspec():
    return pl.BlockSpec((1, 256), lambda i: (0, 0))


def _ln(o, g, be):
    mu = jnp.mean(o, axis=-1, keepdims=True)
    var = jnp.mean((o - mu) * (o - mu), axis=-1, keepdims=True)
    return (o - mu) * lax.rsqrt(var + 1e-5) * g + be


def _silu(t):
    return t * lax.logistic(t)


def _encode_node_body(x_ref, w1, b1, w2, b2, g, be, o_ref):
    t = _silu(jnp.dot(x_ref[...], w1[...], preferred_element_type=F32) + b1[...])
    o = jnp.dot(t, w2[...], preferred_element_type=F32) + b2[...]
    o_ref[...] = _ln(o, g[...], be[...])


def _encode_node(x, w1, b1, w2, b2, g, be):
    n = x.shape[0]
    r = _row_block(n)
    return pl.pallas_call(
        _encode_node_body,
        grid=(n // r,),
        in_specs=[
            pl.BlockSpec((r, 256), lambda i: (i, 0)),
            _wspec(), _bspec(), _wspec(), _bspec(), _bspec(), _bspec(),
        ],
        out_specs=pl.BlockSpec((r, 256), lambda i: (i, 0)),
        out_shape=jax.ShapeDtypeStruct((n, 256), F32),
        compiler_params=pltpu.CompilerParams(
            dimension_semantics=("parallel",)),
    )(x, w1, b1, w2, b2, g, be)


def _encode_edge_body(a_ref, w1, b1, w2, b2, g, be, o_ref):
    t = _silu(jnp.dot(a_ref[...], w1[...], preferred_element_type=F32) + b1[...])
    o = jnp.dot(t, w2[...], preferred_element_type=F32) + b2[...]
    o_ref[...] = _ln(o, g[...], be[...])


def _encode_edge(a, w1, b1, w2, b2, g, be):
    e, de = a.shape
    r = _row_block(e)
    return pl.pallas_call(
        _encode_edge_body,
        grid=(e // r,),
        in_specs=[
            pl.BlockSpec((r, de), lambda i: (i, 0)),
            pl.BlockSpec((de, 256), lambda i: (0, 0)),
            _bspec(), _wspec(), _bspec(), _bspec(), _bspec(),
        ],
        out_specs=pl.BlockSpec((r, 256), lambda i: (i, 0)),
        out_shape=jax.ShapeDtypeStruct((e, 256), F32),
        compiler_params=pltpu.CompilerParams(
            dimension_semantics=("parallel",)),
    )(a, w1, b1, w2, b2, g, be)


def _pack_bf16_pair(o):
    k = o.shape[-1] // 2
    a = lax.bitcast_convert_type(
        o[:, :k].astype(jnp.bfloat16), jnp.uint16).astype(jnp.int32)
    b = lax.bitcast_convert_type(
        o[:, k:].astype(jnp.bfloat16), jnp.uint16).astype(jnp.int32)
    return a | lax.shift_left(b, 16)


def _unpack_bf16_pair(w):
    lo = lax.bitcast_convert_type(
        (w & 0xFFFF).astype(jnp.uint16), jnp.bfloat16).astype(F32)
    hi = lax.bitcast_convert_type(
        lax.shift_right_logical(w, 16).astype(jnp.uint16),
        jnp.bfloat16).astype(F32)
    return jnp.concatenate([lo, hi], axis=-1)


def _dual_project_body(h_ref, ws, wd, os_ref, od_ref):
    h = h_ref[...]
    os_ref[...] = _pack_bf16_pair(
        jnp.dot(h, ws[...], preferred_element_type=F32))
    od_ref[...] = _pack_bf16_pair(
        jnp.dot(h, wd[...], preferred_element_type=F32))


def _dual_project(h, ws, wd):
    n = h.shape[0]
    r = _row_block(n)
    return pl.pallas_call(
        _dual_project_body,
        grid=(n // r,),
        in_specs=[pl.BlockSpec((r, 256), lambda i: (i, 0)), _wspec(), _wspec()],
        out_specs=[pl.BlockSpec((r, 128), lambda i: (i, 0))] * 2,
        out_shape=[jax.ShapeDtypeStruct((n, 128), jnp.int32)] * 2,
        compiler_params=pltpu.CompilerParams(
            dimension_semantics=("parallel",)),
    )(h, ws, wd)


def _edge_update_body(e_ref, gs_ref, gd_ref, w1, b1, w2, b2, g, be, o_ref):
    t = jnp.dot(e_ref[...], w1[...], preferred_element_type=F32)
    gg = _unpack_bf16_pair(gs_ref[...]) + _unpack_bf16_pair(gd_ref[...])
    t = _silu(t + gg + b1[...])
    o = jnp.dot(t, w2[...], preferred_element_type=F32) + b2[...]
    o_ref[...] = e_ref[...] + _ln(o, g[...], be[...])


def _edge_update(e, gs, gd, w1, b1, w2, b2, g, be):
    n = e.shape[0]
    r = _row_block(n)
    rspec = pl.BlockSpec((r, 256), lambda i: (i, 0))
    gspec = pl.BlockSpec((r, 128), lambda i: (i, 0))
    return pl.pallas_call(
        _edge_update_body,
        grid=(n // r,),
        in_specs=[
            rspec, gspec, gspec,
            _wspec(), _bspec(), _wspec(), _bspec(), _bspec(), _bspec(),
        ],
        out_specs=rspec,
        out_shape=jax.ShapeDtypeStruct((n, 256), F32),
        compiler_params=pltpu.CompilerParams(
            dimension_semantics=("parallel",)),
    )(e, gs, gd, w1, b1, w2, b2, g, be)


def _node_update_body(h_ref, a_ref, w1h, w1a, b1, w2, b2, g, be, o_ref):
    t = (jnp.dot(h_ref[...], w1h[...], preferred_element_type=F32)
         + jnp.dot(a_ref[...], w1a[...], preferred_element_type=F32))
    t = _silu(t + b1[...])
    o = jnp.dot(t, w2[...], preferred_element_type=F32) + b2[...]
    o_ref[...] = h_ref[...] + _ln(o, g[...], be[...])


def _node_update(h, agg, w1h, w1a, b1, w2, b2, g, be):
    n = h.shape[0]
    r = _row_block(n)
    return pl.pallas_call(
        _node_update_body,
        grid=(n // r,),
        in_specs=[
            pl.BlockSpec((r, 256), lambda i: (i, 0)),
            pl.BlockSpec((r, 256), lambda i: (i, 0)),
            _wspec(), _wspec(), _bspec(), _wspec(), _bspec(), _bspec(), _bspec(),
        ],
        out_specs=pl.BlockSpec((r, 256), lambda i: (i, 0)),
        out_shape=jax.ShapeDtypeStruct((n, 256), F32),
        compiler_params=pltpu.CompilerParams(
            dimension_semantics=("parallel",)),
    )(h, agg, w1h, w1a, b1, w2, b2, g, be)


def _decode_body(h_ref, w1, b1, w2, b2, o_ref):
    t = _silu(jnp.dot(h_ref[...], w1[...], preferred_element_type=F32) + b1[...])
    o_ref[...] = jnp.dot(t, w2[...], preferred_element_type=F32) + b2[...]


def _decode(h, w1, b1, w2, b2):
    n = h.shape[0]
    r = _row_block(n)
    return pl.pallas_call(
        _decode_body,
        grid=(n // r,),
        in_specs=[
            pl.BlockSpec((r, 256), lambda i: (i, 0)),
            _wspec(), _bspec(), _wspec(), _bspec(),
        ],
        out_specs=pl.BlockSpec((r, 256), lambda i: (i, 0)),
        out_shape=jax.ShapeDtypeStruct((n, 256), F32),
        compiler_params=pltpu.CompilerParams(
            dimension_semantics=("parallel",)),
    )(h, w1, b1, w2, b2)


_NC = 2
_NS = 16
_GK = 200
_SK = 80
_WB = 80


def _sc_gather(hs, hd, src, dst):
    n_e = src.shape[0]
    dw = hs.shape[1]
    per_w = n_e // (_NC * _NS)
    nchunk = per_w // _GK
    mesh = plsc.VectorSubcoreMesh(core_axis_name="c", subcore_axis_name="s",
                                  num_cores=_NC, num_subcores=_NS)

    @functools.partial(
        pl.kernel,
        out_type=(jax.ShapeDtypeStruct((n_e, dw), jnp.int32),
                  jax.ShapeDtypeStruct((n_e, dw), jnp.int32)),
        mesh=mesh,
        scratch_types=[
            [pltpu.VMEM((_GK,), jnp.int32)] * 2,
            [pltpu.VMEM((_GK,), jnp.int32)] * 2,
            [pltpu.VMEM((_GK, dw), jnp.int32)] * 2,
            [pltpu.VMEM((_GK, dw), jnp.int32)] * 2,
            [pltpu.SemaphoreType.DMA] * 2,
            [pltpu.SemaphoreType.DMA] * 2,
        ])
    def k(hs_hbm, hd_hbm, src_hbm, dst_hbm, gs_hbm, gd_hbm,
          si, di, bs, bd, ss, sd):
        wid = lax.axis_index("s") * _NC + lax.axis_index("c")
        base = wid * per_w

        def start(j, b):
            off = base + j * _GK
            pltpu.sync_copy(src_hbm.at[pl.ds(off, _GK)], si[b])
            pltpu.sync_copy(dst_hbm.at[pl.ds(off, _GK)], di[b])
            pltpu.async_copy(hs_hbm.at[si[b]], bs[b], ss[b])
            pltpu.async_copy(hd_hbm.at[di[b]], bd[b], sd[b])

        def finish(j, b):
            off = base + j * _GK
            pltpu.make_async_copy(hs_hbm.at[si[b]], bs[b], ss[b]).wait()
            pltpu.make_async_copy(hd_hbm.at[di[b]], bd[b], sd[b]).wait()
            pltpu.sync_copy(bs[b], gs_hbm.at[pl.ds(off, _GK)])
            pltpu.sync_copy(bd[b], gd_hbm.at[pl.ds(off, _GK)])

        start(0, 0)
        if nchunk > 1:
            start(1, 1)

        def body(g, carry):
            j0 = 2 * g
            j1 = j0 + 1
            finish(j0, 0)

            @pl.when(j0 + 2 < nchunk)
            def _():
                start(j0 + 2, 0)

            @pl.when(j1 < nchunk)
            def _():
                finish(j1, 1)

            @pl.when(j1 + 2 < nchunk)
            def _():
                start(j1 + 2, 1)

            return carry

        lax.fori_loop(0, (nchunk + 1) // 2, body, 0)

    return k(hs, hd, src, dst)


def _sc_segsum(e, dst, n):
    n_e, d = e.shape
    dh = d // 2
    per_tile = n_e // _NS
    nchunk = per_tile // _SK
    npad = _NS * _WB * ((n + _NS * _WB - 1) // (_NS * _WB))
    rows_per_tile = npad // _NS
    nwb = rows_per_tile // _WB
    mesh = plsc.VectorSubcoreMesh(core_axis_name="c", subcore_axis_name="s",
                                  num_cores=_NC, num_subcores=_NS)

    @functools.partial(
        pl.kernel,
        out_type=jax.ShapeDtypeStruct((npad, d), F32),
        mesh=mesh,
        scratch_types=[
            [pltpu.VMEM((_SK,), jnp.int32)] * 2,
            [pltpu.VMEM((_SK, dh), F32)] * 2,
            pltpu.VMEM((_WB, dh), F32),
            pltpu.VMEM_SHARED((npad, dh), F32),
            [pltpu.SemaphoreType.DMA] * 2,
            [pltpu.SemaphoreType.DMA] * 2,
        ])
    def k(e_hbm, dst_hbm, agg_hbm, idxb, ebuf, wbuf, table, six, sro):
        c = lax.axis_index("c")
        s = lax.axis_index("s")
        col0 = c * dh

        zero16 = jnp.zeros((16,), F32)

        def zrow(r, carry):
            for jj in range(dh // 16):
                wbuf[r, pl.ds(jj * 16, 16)] = zero16
            return carry

        lax.fori_loop(0, _WB, zrow, 0)

        def ztab(t, carry):
            pltpu.sync_copy(
                wbuf, table.at[pl.ds(s * rows_per_tile + t * _WB, _WB)])
            return carry

        lax.fori_loop(0, nwb, ztab, 0)
        plsc.subcore_barrier()

        def start(j, b):
            off = s * per_tile + j * _SK
            pltpu.async_copy(dst_hbm.at[pl.ds(off, _SK)], idxb[b], six[b])
            pltpu.async_copy(e_hbm.at[pl.ds(off, _SK), pl.ds(col0, dh)],
                             ebuf[b], sro[b])

        def finish(j, b):
            off = s * per_tile + j * _SK
            pltpu.make_async_copy(
                dst_hbm.at[pl.ds(off, _SK)], idxb[b], six[b]).wait()
            pltpu.make_async_copy(
                e_hbm.at[pl.ds(off, _SK), pl.ds(col0, dh)],
                ebuf[b], sro[b]).wait()
            pltpu.sync_copy(ebuf[b], table.at[idxb[b]], add=True)

        start(0, 0)
        if nchunk > 1:
            start(1, 1)

        def body(g, carry):
            j0 = 2 * g
            j1 = j0 + 1
            finish(j0, 0)

            @pl.when(j0 + 2 < nchunk)
            def _():
                start(j0 + 2, 0)

            @pl.when(j1 < nchunk)
            def _():
                finish(j1, 1)

            @pl.when(j1 + 2 < nchunk)
            def _():
                start(j1 + 2, 1)

            return carry

        lax.fori_loop(0, (nchunk + 1) // 2, body, 0)
        plsc.subcore_barrier()

        def wb(t, carry):
            r0 = s * rows_per_tile + t * _WB
            pltpu.sync_copy(table.at[pl.ds(r0, _WB)], wbuf)
            pltpu.sync_copy(wbuf, agg_hbm.at[pl.ds(r0, _WB),
                                             pl.ds(col0, dh)])
            return carry

        lax.fori_loop(0, nwb, wb, 0)

    return k(e, dst)


def kernel(x, edge_index, edge_attr,
           ne_W1, ne_b1, ne_W2, ne_b2, ne_g, ne_be,
           ee_W1, ee_b1, ee_W2, ee_b2, ee_g, ee_be,
           pe_W1, pe_b1, pe_W2, pe_b2, pe_g, pe_be,
           pn_W1, pn_b1, pn_W2, pn_b2, pn_g, pn_be,
           de_W1, de_b1, de_W2, de_b2):
    n, d = x.shape
    num_layers = pe_W1.shape[0]
    src = edge_index[0]
    dst = edge_index[1]

    r1 = lambda b: b.reshape(1, -1)

    h = _encode_node(x, ne_W1, r1(ne_b1), ne_W2, r1(ne_b2), r1(ne_g), r1(ne_be))
    e = _encode_edge(edge_attr, ee_W1, r1(ee_b1), ee_W2, r1(ee_b2),
                     r1(ee_g), r1(ee_be))

    for i in range(num_layers):
        w1 = pe_W1[i]
        hs, hd = _dual_project(h, w1[d:2 * d], w1[2 * d:])
        gs, gd = _sc_gather(hs, hd, src, dst)
        e = _edge_update(e, gs, gd, w1[:d], r1(pe_b1[i]), pe_W2[i],
                         r1(pe_b2[i]), r1(pe_g[i]), r1(pe_be[i]))
        agg = _sc_segsum(e, dst, n)
        h = _node_update(h, agg, pn_W1[i][:d], pn_W1[i][d:], r1(pn_b1[i]),
                         pn_W2[i], r1(pn_b2[i]), r1(pn_g[i]), r1(pn_be[i]))

    return _decode(h, de_W1, r1(de_b1), de_W2, r1(de_b2))


# speedup vs baseline: 1.2374x; 1.0098x over previous
import functools

import jax
import jax.numpy as jnp
from jax import lax
from jax.experimental import pallas as pl
from jax.experimental.pallas import tpu as pltpu
from jax.experimental.pallas import tpu_sc as plsc

F32 = jnp.float32


def _row_block(n, target=2048):
    best = 8
    for r in range(8, target + 1, 8):
        if n % r == 0:
            best = r
    return best


def _wspec():
    return pl.BlockSpec((256, 256), lambda i: (0, 0))


def _bspec():
    return pl.BlockSpec((1, 256), lambda i: (0, 0))


def _ln(o, g, be):
    mu = jnp.mean(o, axis=-1, keepdims=True)
    var = jnp.mean((o - mu) * (o - mu), axis=-1, keepdims=True)
    return (o - mu) * lax.rsqrt(var + 1e-5) * g + be


def _silu(t):
    return t * lax.logistic(t)


def _encode_node_body(x_ref, w1, b1, w2, b2, g, be, ws, wd,
                      o_ref, os_ref, od_ref):
    t = _silu(jnp.dot(x_ref[...], w1[...], preferred_element_type=F32) + b1[...])
    o = jnp.dot(t, w2[...], preferred_element_type=F32) + b2[...]
    h = _ln(o, g[...], be[...])
    o_ref[...] = h
    os_ref[...] = _pack_bf16_pair(jnp.dot(h, ws[...],
                                          preferred_element_type=F32))
    od_ref[...] = _pack_bf16_pair(jnp.dot(h, wd[...],
                                          preferred_element_type=F32))


def _encode_node(x, w1, b1, w2, b2, g, be, ws, wd):
    n = x.shape[0]
    r = _row_block(n)
    return pl.pallas_call(
        _encode_node_body,
        grid=(n // r,),
        in_specs=[
            pl.BlockSpec((r, 256), lambda i: (i, 0)),
            _wspec(), _bspec(), _wspec(), _bspec(), _bspec(), _bspec(),
            _wspec(), _wspec(),
        ],
        out_specs=[pl.BlockSpec((r, 256), lambda i: (i, 0)),
                   pl.BlockSpec((r, 128), lambda i: (i, 0)),
                   pl.BlockSpec((r, 128), lambda i: (i, 0))],
        out_shape=[jax.ShapeDtypeStruct((n, 256), F32),
                   jax.ShapeDtypeStruct((n, 128), jnp.int32),
                   jax.ShapeDtypeStruct((n, 128), jnp.int32)],
        compiler_params=pltpu.CompilerParams(
            dimension_semantics=("parallel",)),
    )(x, w1, b1, w2, b2, g, be, ws, wd)


def _encode_edge_body(a_ref, w1, b1, w2, b2, g, be, o_ref):
    t = _silu(jnp.dot(a_ref[...], w1[...], preferred_element_type=F32) + b1[...])
    o = jnp.dot(t.astype(jnp.bfloat16), w2[...],
                preferred_element_type=F32) + b2[...]
    o_ref[...] = _ln(o, g[...], be[...])


def _encode_edge(a, w1, b1, w2, b2, g, be):
    e, de = a.shape
    r = _row_block(e)
    return pl.pallas_call(
        _encode_edge_body,
        grid=(e // r,),
        in_specs=[
            pl.BlockSpec((r, de), lambda i: (i, 0)),
            pl.BlockSpec((de, 256), lambda i: (0, 0)),
            _bspec(), _wspec(), _bspec(), _bspec(), _bspec(),
        ],
        out_specs=pl.BlockSpec((r, 256), lambda i: (i, 0)),
        out_shape=jax.ShapeDtypeStruct((e, 256), F32),
        compiler_params=pltpu.CompilerParams(
            dimension_semantics=("parallel",)),
    )(a, w1, b1, w2, b2, g, be)


def _pack_bf16_pair(o):
    k = o.shape[-1] // 2
    a = lax.bitcast_convert_type(
        o[:, :k].astype(jnp.bfloat16), jnp.uint16).astype(jnp.int32)
    b = lax.bitcast_convert_type(
        o[:, k:].astype(jnp.bfloat16), jnp.uint16).astype(jnp.int32)
    return a | lax.shift_left(b, 16)


def _unpack_bf16_pair(w):
    lo = lax.bitcast_convert_type(
        (w & 0xFFFF).astype(jnp.uint16), jnp.bfloat16).astype(F32)
    hi = lax.bitcast_convert_type(
        lax.shift_right_logical(w, 16).astype(jnp.uint16),
        jnp.bfloat16).astype(F32)
    return jnp.concatenate([lo, hi], axis=-1)


def _edge_update_body(e_ref, gs_ref, gd_ref, w1, b1, w2, b2, g, be, o_ref):
    e = e_ref[...]
    t = jnp.dot(e.astype(jnp.bfloat16), w1[...], preferred_element_type=F32)
    gg = _unpack_bf16_pair(gs_ref[...]) + _unpack_bf16_pair(gd_ref[...])
    t = _silu(t + gg + b1[...])
    o = jnp.dot(t.astype(jnp.bfloat16), w2[...],
                preferred_element_type=F32) + b2[...]
    o_ref[...] = e + _ln(o, g[...], be[...])


def _edge_update(e, gs, gd, w1, b1, w2, b2, g, be):
    n = e.shape[0]
    r = _row_block(n)
    rspec = pl.BlockSpec((r, 256), lambda i: (i, 0))
    gspec = pl.BlockSpec((r, 128), lambda i: (i, 0))
    return pl.pallas_call(
        _edge_update_body,
        grid=(n // r,),
        in_specs=[
            rspec, gspec, gspec,
            _wspec(), _bspec(), _wspec(), _bspec(), _bspec(), _bspec(),
        ],
        out_specs=rspec,
        out_shape=jax.ShapeDtypeStruct((n, 256), F32),
        compiler_params=pltpu.CompilerParams(
            dimension_semantics=("parallel",)),
    )(e, gs, gd, w1, b1, w2, b2, g, be)


def _node_core(h_ref, a_ref, w1h, w1a, b1, w2, b2, g, be):
    t = (jnp.dot(h_ref[...], w1h[...], preferred_element_type=F32)
         + jnp.dot(a_ref[...], w1a[...], preferred_element_type=F32))
    t = _silu(t + b1[...])
    o = jnp.dot(t, w2[...], preferred_element_type=F32) + b2[...]
    return h_ref[...] + _ln(o, g[...], be[...])


def _node_update_body(h_ref, a_ref, w1h, w1a, b1, w2, b2, g, be, o_ref):
    o_ref[...] = _node_core(h_ref, a_ref, w1h, w1a, b1, w2, b2, g, be)


def _node_update_proj_body(h_ref, a_ref, w1h, w1a, b1, w2, b2, g, be,
                           ws, wd, o_ref, os_ref, od_ref):
    h = _node_core(h_ref, a_ref, w1h, w1a, b1, w2, b2, g, be)
    o_ref[...] = h
    os_ref[...] = _pack_bf16_pair(jnp.dot(h, ws[...],
                                          preferred_element_type=F32))
    od_ref[...] = _pack_bf16_pair(jnp.dot(h, wd[...],
                                          preferred_element_type=F32))


def _node_update(h, agg, w1h, w1a, b1, w2, b2, g, be, ws=None, wd=None):
    n = h.shape[0]
    r = _row_block(n)
    rspec = pl.BlockSpec((r, 256), lambda i: (i, 0))
    pspec = pl.BlockSpec((r, 128), lambda i: (i, 0))
    params = pltpu.CompilerParams(dimension_semantics=("parallel",))
    if ws is None:
        return pl.pallas_call(
            _node_update_body,
            grid=(n // r,),
            in_specs=[rspec, rspec, _wspec(), _wspec(), _bspec(), _wspec(),
                      _bspec(), _bspec(), _bspec()],
            out_specs=rspec,
            out_shape=jax.ShapeDtypeStruct((n, 256), F32),
            compiler_params=params,
        )(h, agg, w1h, w1a, b1, w2, b2, g, be)
    return pl.pallas_call(
        _node_update_proj_body,
        grid=(n // r,),
        in_specs=[rspec, rspec, _wspec(), _wspec(), _bspec(), _wspec(),
                  _bspec(), _bspec(), _bspec(), _wspec(), _wspec()],
        out_specs=[rspec, pspec, pspec],
        out_shape=[jax.ShapeDtypeStruct((n, 256), F32),
                   jax.ShapeDtypeStruct((n, 128), jnp.int32),
                   jax.ShapeDtypeStruct((n, 128), jnp.int32)],
        compiler_params=params,
    )(h, agg, w1h, w1a, b1, w2, b2, g, be, ws, wd)


def _decode_body(h_ref, w1, b1, w2, b2, o_ref):
    t = _silu(jnp.dot(h_ref[...], w1[...], preferred_element_type=F32) + b1[...])
    o_ref[...] = jnp.dot(t, w2[...], preferred_element_type=F32) + b2[...]


def _decode(h, w1, b1, w2, b2):
    n = h.shape[0]
    r = _row_block(n)
    return pl.pallas_call(
        _decode_body,
        grid=(n // r,),
        in_specs=[
            pl.BlockSpec((r, 256), lambda i: (i, 0)),
            _wspec(), _bspec(), _wspec(), _bspec(),
        ],
        out_specs=pl.BlockSpec((r, 256), lambda i: (i, 0)),
        out_shape=jax.ShapeDtypeStruct((n, 256), F32),
        compiler_params=pltpu.CompilerParams(
            dimension_semantics=("parallel",)),
    )(h, w1, b1, w2, b2)


_NC = 2
_NS = 16
_GK = 200
_SK = 80
_WB = 80


def _sc_gather(hs, hd, src, dst):
    n_e = src.shape[0]
    dw = hs.shape[1]
    per_w = n_e // (_NC * _NS)
    nchunk = per_w // _GK
    mesh = plsc.VectorSubcoreMesh(core_axis_name="c", subcore_axis_name="s",
                                  num_cores=_NC, num_subcores=_NS)

    @functools.partial(
        pl.kernel,
        out_type=(jax.ShapeDtypeStruct((n_e, dw), jnp.int32),
                  jax.ShapeDtypeStruct((n_e, dw), jnp.int32)),
        mesh=mesh,
        scratch_types=[
            [pltpu.VMEM((_GK,), jnp.int32)] * 2,
            [pltpu.VMEM((_GK,), jnp.int32)] * 2,
            [pltpu.VMEM((_GK, dw), jnp.int32)] * 2,
            [pltpu.VMEM((_GK, dw), jnp.int32)] * 2,
            [pltpu.SemaphoreType.DMA] * 2,
            [pltpu.SemaphoreType.DMA] * 2,
        ])
    def k(hs_hbm, hd_hbm, src_hbm, dst_hbm, gs_hbm, gd_hbm,
          si, di, bs, bd, ss, sd):
        wid = lax.axis_index("s") * _NC + lax.axis_index("c")
        base = wid * per_w

        def start(j, b):
            off = base + j * _GK
            pltpu.sync_copy(src_hbm.at[pl.ds(off, _GK)], si[b])
            pltpu.sync_copy(dst_hbm.at[pl.ds(off, _GK)], di[b])
            pltpu.async_copy(hs_hbm.at[si[b]], bs[b], ss[b])
            pltpu.async_copy(hd_hbm.at[di[b]], bd[b], sd[b])

        def finish(j, b):
            off = base + j * _GK
            pltpu.make_async_copy(hs_hbm.at[si[b]], bs[b], ss[b]).wait()
            pltpu.make_async_copy(hd_hbm.at[di[b]], bd[b], sd[b]).wait()
            pltpu.sync_copy(bs[b], gs_hbm.at[pl.ds(off, _GK)])
            pltpu.sync_copy(bd[b], gd_hbm.at[pl.ds(off, _GK)])

        start(0, 0)
        if nchunk > 1:
            start(1, 1)

        def body(g, carry):
            j0 = 2 * g
            j1 = j0 + 1
            finish(j0, 0)

            @pl.when(j0 + 2 < nchunk)
            def _():
                start(j0 + 2, 0)

            @pl.when(j1 < nchunk)
            def _():
                finish(j1, 1)

            @pl.when(j1 + 2 < nchunk)
            def _():
                start(j1 + 2, 1)

            return carry

        lax.fori_loop(0, (nchunk + 1) // 2, body, 0)

    return k(hs, hd, src, dst)


def _sc_segsum(e, dst, n):
    n_e, d = e.shape
    dh = d // 2
    per_tile = n_e // _NS
    nchunk = per_tile // _SK
    npad = _NS * _WB * ((n + _NS * _WB - 1) // (_NS * _WB))
    rows_per_tile = npad // _NS
    nwb = rows_per_tile // _WB
    mesh = plsc.VectorSubcoreMesh(core_axis_name="c", subcore_axis_name="s",
                                  num_cores=_NC, num_subcores=_NS)

    @functools.partial(
        pl.kernel,
        out_type=jax.ShapeDtypeStruct((npad, d), F32),
        mesh=mesh,
        scratch_types=[
            [pltpu.VMEM((_SK,), jnp.int32)] * 2,
            [pltpu.VMEM((_SK, dh), F32)] * 2,
            pltpu.VMEM((_WB, dh), F32),
            pltpu.VMEM_SHARED((npad, dh), F32),
            [pltpu.SemaphoreType.DMA] * 2,
            [pltpu.SemaphoreType.DMA] * 2,
        ])
    def k(e_hbm, dst_hbm, agg_hbm, idxb, ebuf, wbuf, table, six, sro):
        c = lax.axis_index("c")
        s = lax.axis_index("s")
        col0 = c * dh

        zero16 = jnp.zeros((16,), F32)

        def zrow(r, carry):
            for jj in range(dh // 16):
                wbuf[r, pl.ds(jj * 16, 16)] = zero16
            return carry

        lax.fori_loop(0, _WB, zrow, 0)

        def ztab(t, carry):
            pltpu.sync_copy(
                wbuf, table.at[pl.ds(s * rows_per_tile + t * _WB, _WB)])
            return carry

        lax.fori_loop(0, nwb, ztab, 0)
        plsc.subcore_barrier()

        def start(j, b):
            off = s * per_tile + j * _SK
            pltpu.async_copy(dst_hbm.at[pl.ds(off, _SK)], idxb[b], six[b])
            pltpu.async_copy(e_hbm.at[pl.ds(off, _SK), pl.ds(col0, dh)],
                             ebuf[b], sro[b])

        def finish(j, b):
            off = s * per_tile + j * _SK
            pltpu.make_async_copy(
                dst_hbm.at[pl.ds(off, _SK)], idxb[b], six[b]).wait()
            pltpu.make_async_copy(
                e_hbm.at[pl.ds(off, _SK), pl.ds(col0, dh)],
                ebuf[b], sro[b]).wait()
            pltpu.sync_copy(ebuf[b], table.at[idxb[b]], add=True)

        start(0, 0)
        if nchunk > 1:
            start(1, 1)

        def body(g, carry):
            j0 = 2 * g
            j1 = j0 + 1
            finish(j0, 0)

            @pl.when(j0 + 2 < nchunk)
            def _():
                start(j0 + 2, 0)

            @pl.when(j1 < nchunk)
            def _():
                finish(j1, 1)

            @pl.when(j1 + 2 < nchunk)
            def _():
                start(j1 + 2, 1)

            return carry

        lax.fori_loop(0, (nchunk + 1) // 2, body, 0)
        plsc.subcore_barrier()

        def wb(t, carry):
            r0 = s * rows_per_tile + t * _WB
            pltpu.sync_copy(table.at[pl.ds(r0, _WB)], wbuf)
            pltpu.sync_copy(wbuf, agg_hbm.at[pl.ds(r0, _WB),
                                             pl.ds(col0, dh)])
            return carry

        lax.fori_loop(0, nwb, wb, 0)

    return k(e, dst)


def kernel(x, edge_index, edge_attr,
           ne_W1, ne_b1, ne_W2, ne_b2, ne_g, ne_be,
           ee_W1, ee_b1, ee_W2, ee_b2, ee_g, ee_be,
           pe_W1, pe_b1, pe_W2, pe_b2, pe_g, pe_be,
           pn_W1, pn_b1, pn_W2, pn_b2, pn_g, pn_be,
           de_W1, de_b1, de_W2, de_b2):
    n, d = x.shape
    num_layers = pe_W1.shape[0]
    src = edge_index[0]
    dst = edge_index[1]

    r1 = lambda b: b.reshape(1, -1)
    bf16 = jnp.bfloat16

    h, hs, hd = _encode_node(x, ne_W1, r1(ne_b1), ne_W2, r1(ne_b2),
                             r1(ne_g), r1(ne_be),
                             pe_W1[0][d:2 * d], pe_W1[0][2 * d:])
    e = _encode_edge(edge_attr, ee_W1, r1(ee_b1), ee_W2.astype(bf16),
                     r1(ee_b2), r1(ee_g), r1(ee_be))

    for i in range(num_layers):
        gs, gd = _sc_gather(hs, hd, src, dst)
        e = _edge_update(e, gs, gd, pe_W1[i][:d].astype(bf16), r1(pe_b1[i]),
                         pe_W2[i].astype(bf16), r1(pe_b2[i]),
                         r1(pe_g[i]), r1(pe_be[i]))
        agg = _sc_segsum(e, dst, n)
        if i + 1 < num_layers:
            h, hs, hd = _node_update(
                h, agg, pn_W1[i][:d], pn_W1[i][d:], r1(pn_b1[i]),
                pn_W2[i], r1(pn_b2[i]), r1(pn_g[i]), r1(pn_be[i]),
                pe_W1[i + 1][d:2 * d], pe_W1[i + 1][2 * d:])
        else:
            h = _node_update(
                h, agg, pn_W1[i][:d], pn_W1[i][d:], r1(pn_b1[i]),
                pn_W2[i], r1(pn_b2[i]), r1(pn_g[i]), r1(pn_be[i]))

    return _decode(h, de_W1, r1(de_b1), de_W2, r1(de_b2))


# speedup vs baseline: 1.3273x; 1.0726x over previous
import functools
import math

import jax
import jax.numpy as jnp
from jax import lax
from jax.experimental import pallas as pl
from jax.experimental.pallas import tpu as pltpu
from jax.experimental.pallas import tpu_sc as plsc

F32 = jnp.float32


def _row_block(n, target=2048):
    best = 8
    for r in range(8, target + 1, 8):
        if n % r == 0:
            best = r
    return best


def _wspec():
    return pl.BlockSpec((256, 256), lambda i: (0, 0))


def _bspec():
    return pl.BlockSpec((1, 256), lambda i: (0, 0))


def _ln(o, g, be):
    mu = jnp.mean(o, axis=-1, keepdims=True)
    var = jnp.mean((o - mu) * (o - mu), axis=-1, keepdims=True)
    return (o - mu) * lax.rsqrt(var + 1e-5) * g + be


def _silu(t):
    return t * lax.logistic(t)


def _encode_node_body(x_ref, w1, b1, w2, b2, g, be, ws, wd,
                      o_ref, os_ref, od_ref):
    t = _silu(jnp.dot(x_ref[...], w1[...], preferred_element_type=F32) + b1[...])
    o = jnp.dot(t, w2[...], preferred_element_type=F32) + b2[...]
    h = _ln(o, g[...], be[...])
    o_ref[...] = h
    os_ref[...] = _pack_bf16_pair(jnp.dot(h, ws[...],
                                          preferred_element_type=F32))
    od_ref[...] = _pack_bf16_pair(jnp.dot(h, wd[...],
                                          preferred_element_type=F32))


def _encode_node(x, w1, b1, w2, b2, g, be, ws, wd):
    n = x.shape[0]
    r = _row_block(n)
    return pl.pallas_call(
        _encode_node_body,
        grid=(n // r,),
        in_specs=[
            pl.BlockSpec((r, 256), lambda i: (i, 0)),
            _wspec(), _bspec(), _wspec(), _bspec(), _bspec(), _bspec(),
            _wspec(), _wspec(),
        ],
        out_specs=[pl.BlockSpec((r, 256), lambda i: (i, 0)),
                   pl.BlockSpec((r, 128), lambda i: (i, 0)),
                   pl.BlockSpec((r, 128), lambda i: (i, 0))],
        out_shape=[jax.ShapeDtypeStruct((n, 256), F32),
                   jax.ShapeDtypeStruct((n, 128), jnp.int32),
                   jax.ShapeDtypeStruct((n, 128), jnp.int32)],
        compiler_params=pltpu.CompilerParams(
            dimension_semantics=("parallel",)),
    )(x, w1, b1, w2, b2, g, be, ws, wd)


def _encode_edge_body(a_ref, w1, b1, w2, b2, g, be, o_ref):
    t = _silu(jnp.dot(a_ref[...], w1[...], preferred_element_type=F32) + b1[...])
    o = jnp.dot(t.astype(jnp.bfloat16), w2[...],
                preferred_element_type=F32) + b2[...]
    o_ref[...] = _ln(o, g[...], be[...])


def _encode_edge(a, w1, b1, w2, b2, g, be, rows, off):
    de = a.shape[1]
    r = _row_block(math.gcd(rows, off) if off else rows)
    nblk = off // r
    return pl.pallas_call(
        _encode_edge_body,
        grid=(rows // r,),
        in_specs=[
            pl.BlockSpec((r, de), lambda i: (i + nblk, 0)),
            pl.BlockSpec((de, 256), lambda i: (0, 0)),
            _bspec(), _wspec(), _bspec(), _bspec(), _bspec(),
        ],
        out_specs=pl.BlockSpec((r, 256), lambda i: (i, 0)),
        out_shape=jax.ShapeDtypeStruct((rows, 256), F32),
        compiler_params=pltpu.CompilerParams(
            dimension_semantics=("parallel",)),
    )(a, w1, b1, w2, b2, g, be)


def _pack_bf16_pair(o):
    k = o.shape[-1] // 2
    a = lax.bitcast_convert_type(
        o[:, :k].astype(jnp.bfloat16), jnp.uint16).astype(jnp.int32)
    b = lax.bitcast_convert_type(
        o[:, k:].astype(jnp.bfloat16), jnp.uint16).astype(jnp.int32)
    return a | lax.shift_left(b, 16)


def _unpack_bf16_pair(w):
    lo = lax.bitcast_convert_type(
        (w & 0xFFFF).astype(jnp.uint16), jnp.bfloat16).astype(F32)
    hi = lax.bitcast_convert_type(
        lax.shift_right_logical(w, 16).astype(jnp.uint16),
        jnp.bfloat16).astype(F32)
    return jnp.concatenate([lo, hi], axis=-1)


def _edge_update_body(e_ref, gs_ref, gd_ref, w1, b1, w2, b2, g, be, o_ref):
    e = e_ref[...]
    t = jnp.dot(e.astype(jnp.bfloat16), w1[...], preferred_element_type=F32)
    gg = _unpack_bf16_pair(gs_ref[...]) + _unpack_bf16_pair(gd_ref[...])
    t = _silu(t + gg + b1[...])
    o = jnp.dot(t.astype(jnp.bfloat16), w2[...],
                preferred_element_type=F32) + b2[...]
    o_ref[...] = e + _ln(o, g[...], be[...])


def _edge_update(e, gs, gd, w1, b1, w2, b2, g, be):
    n = e.shape[0]
    r = _row_block(n)
    rspec = pl.BlockSpec((r, 256), lambda i: (i, 0))
    gspec = pl.BlockSpec((r, 128), lambda i: (i, 0))
    return pl.pallas_call(
        _edge_update_body,
        grid=(n // r,),
        in_specs=[
            rspec, gspec, gspec,
            _wspec(), _bspec(), _wspec(), _bspec(), _bspec(), _bspec(),
        ],
        out_specs=rspec,
        out_shape=jax.ShapeDtypeStruct((n, 256), F32),
        compiler_params=pltpu.CompilerParams(
            dimension_semantics=("parallel",)),
    )(e, gs, gd, w1, b1, w2, b2, g, be)


def _node_core(h_ref, a1_ref, a2_ref, w1h, w1a, b1, w2, b2, g, be):
    agg = a1_ref[...] + a2_ref[...]
    t = (jnp.dot(h_ref[...], w1h[...], preferred_element_type=F32)
         + jnp.dot(agg, w1a[...], preferred_element_type=F32))
    t = _silu(t + b1[...])
    o = jnp.dot(t, w2[...], preferred_element_type=F32) + b2[...]
    return h_ref[...] + _ln(o, g[...], be[...])


def _node_update_body(h_ref, a1_ref, a2_ref, w1h, w1a, b1, w2, b2, g, be,
                      o_ref):
    o_ref[...] = _node_core(h_ref, a1_ref, a2_ref, w1h, w1a, b1, w2, b2, g, be)


def _node_update_proj_body(h_ref, a1_ref, a2_ref, w1h, w1a, b1, w2, b2, g, be,
                           ws, wd, o_ref, os_ref, od_ref):
    h = _node_core(h_ref, a1_ref, a2_ref, w1h, w1a, b1, w2, b2, g, be)
    o_ref[...] = h
    os_ref[...] = _pack_bf16_pair(jnp.dot(h, ws[...],
                                          preferred_element_type=F32))
    od_ref[...] = _pack_bf16_pair(jnp.dot(h, wd[...],
                                          preferred_element_type=F32))


def _node_update(h, agg1, agg2, w1h, w1a, b1, w2, b2, g, be,
                 ws=None, wd=None):
    n = h.shape[0]
    r = _row_block(n)
    rspec = pl.BlockSpec((r, 256), lambda i: (i, 0))
    pspec = pl.BlockSpec((r, 128), lambda i: (i, 0))
    params = pltpu.CompilerParams(dimension_semantics=("parallel",))
    if ws is None:
        return pl.pallas_call(
            _node_update_body,
            grid=(n // r,),
            in_specs=[rspec, rspec, rspec, _wspec(), _wspec(), _bspec(),
                      _wspec(), _bspec(), _bspec(), _bspec()],
            out_specs=rspec,
            out_shape=jax.ShapeDtypeStruct((n, 256), F32),
            compiler_params=params,
        )(h, agg1, agg2, w1h, w1a, b1, w2, b2, g, be)
    return pl.pallas_call(
        _node_update_proj_body,
        grid=(n // r,),
        in_specs=[rspec, rspec, rspec, _wspec(), _wspec(), _bspec(),
                  _wspec(), _bspec(), _bspec(), _bspec(), _wspec(), _wspec()],
        out_specs=[rspec, pspec, pspec],
        out_shape=[jax.ShapeDtypeStruct((n, 256), F32),
                   jax.ShapeDtypeStruct((n, 128), jnp.int32),
                   jax.ShapeDtypeStruct((n, 128), jnp.int32)],
        compiler_params=params,
    )(h, agg1, agg2, w1h, w1a, b1, w2, b2, g, be, ws, wd)


def _decode_body(h_ref, w1, b1, w2, b2, o_ref):
    t = _silu(jnp.dot(h_ref[...], w1[...], preferred_element_type=F32) + b1[...])
    o_ref[...] = jnp.dot(t, w2[...], preferred_element_type=F32) + b2[...]


def _decode(h, w1, b1, w2, b2):
    n = h.shape[0]
    r = _row_block(n)
    return pl.pallas_call(
        _decode_body,
        grid=(n // r,),
        in_specs=[
            pl.BlockSpec((r, 256), lambda i: (i, 0)),
            _wspec(), _bspec(), _wspec(), _bspec(),
        ],
        out_specs=pl.BlockSpec((r, 256), lambda i: (i, 0)),
        out_shape=jax.ShapeDtypeStruct((n, 256), F32),
        compiler_params=pltpu.CompilerParams(
            dimension_semantics=("parallel",)),
    )(h, w1, b1, w2, b2)


_NC = 2
_NS = 16
_WB = 80


def _chunk(per, cap=200):
    best = 8
    for dcand in range(8, cap + 1, 8):
        if per % dcand == 0:
            best = dcand
    return best


def _edge_split(n_e):
    half = n_e // 2
    lo = max(256, ((half - 10240) // 256) * 256)
    best, best_score = None, -1
    for ea in range(lo, half + 10240 + 1, 256):
        if ea <= 0 or ea >= n_e or (n_e - ea) % 256:
            continue
        score = min(_chunk(ea // 32), _chunk((n_e - ea) // 32))
        if score > best_score:
            best, best_score = ea, score
    return best


def _sc_gather(hs, hd, src, dst):
    n_e = src.shape[0]
    dw = hs.shape[1]
    per_w = n_e // (_NC * _NS)
    gk = _chunk(per_w)
    nchunk = per_w // gk
    mesh = plsc.VectorSubcoreMesh(core_axis_name="c", subcore_axis_name="s",
                                  num_cores=_NC, num_subcores=_NS)

    @functools.partial(
        pl.kernel,
        out_type=(jax.ShapeDtypeStruct((n_e, dw), jnp.int32),
                  jax.ShapeDtypeStruct((n_e, dw), jnp.int32)),
        mesh=mesh,
        scratch_types=[
            [pltpu.VMEM((gk,), jnp.int32)] * 2,
            [pltpu.VMEM((gk,), jnp.int32)] * 2,
            [pltpu.VMEM((gk, dw), jnp.int32)] * 2,
            [pltpu.VMEM((gk, dw), jnp.int32)] * 2,
            [pltpu.SemaphoreType.DMA] * 2,
            [pltpu.SemaphoreType.DMA] * 2,
        ])
    def k(hs_hbm, hd_hbm, src_hbm, dst_hbm, gs_hbm, gd_hbm,
          si, di, bs, bd, ss, sd):
        wid = lax.axis_index("s") * _NC + lax.axis_index("c")
        base = wid * per_w

        def start(j, b):
            off = base + j * gk
            pltpu.sync_copy(src_hbm.at[pl.ds(off, gk)], si[b])
            pltpu.sync_copy(dst_hbm.at[pl.ds(off, gk)], di[b])
            pltpu.async_copy(hs_hbm.at[si[b]], bs[b], ss[b])
            pltpu.async_copy(hd_hbm.at[di[b]], bd[b], sd[b])

        def finish(j, b):
            off = base + j * gk
            pltpu.make_async_copy(hs_hbm.at[si[b]], bs[b], ss[b]).wait()
            pltpu.make_async_copy(hd_hbm.at[di[b]], bd[b], sd[b]).wait()
            pltpu.sync_copy(bs[b], gs_hbm.at[pl.ds(off, gk)])
            pltpu.sync_copy(bd[b], gd_hbm.at[pl.ds(off, gk)])

        start(0, 0)
        if nchunk > 1:
            start(1, 1)

        def body(g, carry):
            j0 = 2 * g
            j1 = j0 + 1
            finish(j0, 0)

            @pl.when(j0 + 2 < nchunk)
            def _():
                start(j0 + 2, 0)

            @pl.when(j1 < nchunk)
            def _():
                finish(j1, 1)

            @pl.when(j1 + 2 < nchunk)
            def _():
                start(j1 + 2, 1)

            return carry

        lax.fori_loop(0, (nchunk + 1) // 2, body, 0)

    return k(hs, hd, src, dst)


def _sc_segsum(e, dst, n):
    n_e, d = e.shape
    dh = d // 2
    per_tile = n_e // _NS
    sk = _chunk(per_tile, cap=80)
    nchunk = per_tile // sk
    npad = _NS * _WB * ((n + _NS * _WB - 1) // (_NS * _WB))
    rows_per_tile = npad // _NS
    nwb = rows_per_tile // _WB
    mesh = plsc.VectorSubcoreMesh(core_axis_name="c", subcore_axis_name="s",
                                  num_cores=_NC, num_subcores=_NS)

    @functools.partial(
        pl.kernel,
        out_type=jax.ShapeDtypeStruct((npad, d), F32),
        mesh=mesh,
        scratch_types=[
            [pltpu.VMEM((sk,), jnp.int32)] * 2,
            [pltpu.VMEM((sk, dh), F32)] * 2,
            pltpu.VMEM((_WB, dh), F32),
            pltpu.VMEM_SHARED((npad, dh), F32),
            [pltpu.SemaphoreType.DMA] * 2,
            [pltpu.SemaphoreType.DMA] * 2,
        ])
    def k(e_hbm, dst_hbm, agg_hbm, idxb, ebuf, wbuf, table, six, sro):
        c = lax.axis_index("c")
        s = lax.axis_index("s")
        col0 = c * dh

        zero16 = jnp.zeros((16,), F32)

        def zrow(r, carry):
            for jj in range(dh // 16):
                wbuf[r, pl.ds(jj * 16, 16)] = zero16
            return carry

        lax.fori_loop(0, _WB, zrow, 0)

        def ztab(t, carry):
            pltpu.sync_copy(
                wbuf, table.at[pl.ds(s * rows_per_tile + t * _WB, _WB)])
            return carry

        lax.fori_loop(0, nwb, ztab, 0)
        plsc.subcore_barrier()

        def start(j, b):
            off = s * per_tile + j * sk
            pltpu.async_copy(dst_hbm.at[pl.ds(off, sk)], idxb[b], six[b])
            pltpu.async_copy(e_hbm.at[pl.ds(off, sk), pl.ds(col0, dh)],
                             ebuf[b], sro[b])

        def finish(j, b):
            off = s * per_tile + j * sk
            pltpu.make_async_copy(
                dst_hbm.at[pl.ds(off, sk)], idxb[b], six[b]).wait()
            pltpu.make_async_copy(
                e_hbm.at[pl.ds(off, sk), pl.ds(col0, dh)],
                ebuf[b], sro[b]).wait()
            pltpu.sync_copy(ebuf[b], table.at[idxb[b]], add=True)

        start(0, 0)
        if nchunk > 1:
            start(1, 1)

        def body(g, carry):
            j0 = 2 * g
            j1 = j0 + 1
            finish(j0, 0)

            @pl.when(j0 + 2 < nchunk)
            def _():
                start(j0 + 2, 0)

            @pl.when(j1 < nchunk)
            def _():
                finish(j1, 1)

            @pl.when(j1 + 2 < nchunk)
            def _():
                start(j1 + 2, 1)

            return carry

        lax.fori_loop(0, (nchunk + 1) // 2, body, 0)
        plsc.subcore_barrier()

        def wb(t, carry):
            r0 = s * rows_per_tile + t * _WB
            pltpu.sync_copy(table.at[pl.ds(r0, _WB)], wbuf)
            pltpu.sync_copy(wbuf, agg_hbm.at[pl.ds(r0, _WB),
                                             pl.ds(col0, dh)])
            return carry

        lax.fori_loop(0, nwb, wb, 0)

    return k(e, dst)


def kernel(x, edge_index, edge_attr,
           ne_W1, ne_b1, ne_W2, ne_b2, ne_g, ne_be,
           ee_W1, ee_b1, ee_W2, ee_b2, ee_g, ee_be,
           pe_W1, pe_b1, pe_W2, pe_b2, pe_g, pe_be,
           pn_W1, pn_b1, pn_W2, pn_b2, pn_g, pn_be,
           de_W1, de_b1, de_W2, de_b2):
    n, d = x.shape
    num_layers = pe_W1.shape[0]
    src = edge_index[0]
    dst = edge_index[1]

    n_e = src.shape[0]
    r1 = lambda b: b.reshape(1, -1)
    bf16 = jnp.bfloat16

    ea = _edge_split(n_e)
    src_h = (src[:ea], src[ea:])
    dst_h = (dst[:ea], dst[ea:])
    offs = (0, ea)
    rows = (ea, n_e - ea)

    h, hs, hd = _encode_node(x, ne_W1, r1(ne_b1), ne_W2, r1(ne_b2),
                             r1(ne_g), r1(ne_be),
                             pe_W1[0][d:2 * d], pe_W1[0][2 * d:])
    e = [
        _encode_edge(edge_attr, ee_W1, r1(ee_b1), ee_W2.astype(bf16),
                     r1(ee_b2), r1(ee_g), r1(ee_be), rows[p], offs[p])
        for p in range(2)
    ]

    for i in range(num_layers):
        agg = [None, None]
        for p in range(2):
            gs, gd = _sc_gather(hs, hd, src_h[p], dst_h[p])
            e[p] = _edge_update(e[p], gs, gd, pe_W1[i][:d].astype(bf16),
                                r1(pe_b1[i]), pe_W2[i].astype(bf16),
                                r1(pe_b2[i]), r1(pe_g[i]), r1(pe_be[i]))
            agg[p] = _sc_segsum(e[p], dst_h[p], n)
        if i + 1 < num_layers:
            h, hs, hd = _node_update(
                h, agg[0], agg[1], pn_W1[i][:d], pn_W1[i][d:], r1(pn_b1[i]),
                pn_W2[i], r1(pn_b2[i]), r1(pn_g[i]), r1(pn_be[i]),
                pe_W1[i + 1][d:2 * d], pe_W1[i + 1][2 * d:])
        else:
            h = _node_update(
                h, agg[0], agg[1], pn_W1[i][:d], pn_W1[i][d:], r1(pn_b1[i]),
                pn_W2[i], r1(pn_b2[i]), r1(pn_g[i]), r1(pn_be[i]))

    return _decode(h, de_W1, r1(de_b1), de_W2, r1(de_b2))


# speedup vs baseline: 1.3592x; 1.0241x over previous
import functools
import math

import jax
import jax.numpy as jnp
from jax import lax
from jax.experimental import pallas as pl
from jax.experimental.pallas import tpu as pltpu
from jax.experimental.pallas import tpu_sc as plsc

F32 = jnp.float32


def _row_block(n, target=2048):
    best = 8
    for r in range(8, target + 1, 8):
        if n % r == 0:
            best = r
    return best


def _wspec():
    return pl.BlockSpec((256, 256), lambda i: (0, 0))


def _bspec():
    return pl.BlockSpec((1, 256), lambda i: (0, 0))


def _ln(o, g, be):
    mu = jnp.mean(o, axis=-1, keepdims=True)
    var = jnp.mean((o - mu) * (o - mu), axis=-1, keepdims=True)
    return (o - mu) * lax.rsqrt(var + 1e-5) * g + be


def _silu(t):
    return t * lax.logistic(t)


def _encode_node_body(x_ref, w1, b1, w2, b2, g, be, ws, wd,
                      o_ref, os_ref, od_ref):
    t = _silu(jnp.dot(x_ref[...], w1[...], preferred_element_type=F32) + b1[...])
    o = jnp.dot(t, w2[...], preferred_element_type=F32) + b2[...]
    h = _ln(o, g[...], be[...])
    o_ref[...] = h
    os_ref[...] = _pack_bf16_pair(jnp.dot(h, ws[...],
                                          preferred_element_type=F32))
    od_ref[...] = _pack_bf16_pair(jnp.dot(h, wd[...],
                                          preferred_element_type=F32))


def _encode_node(x, w1, b1, w2, b2, g, be, ws, wd):
    n = x.shape[0]
    r = _row_block(n)
    return pl.pallas_call(
        _encode_node_body,
        grid=(n // r,),
        in_specs=[
            pl.BlockSpec((r, 256), lambda i: (i, 0)),
            _wspec(), _bspec(), _wspec(), _bspec(), _bspec(), _bspec(),
            _wspec(), _wspec(),
        ],
        out_specs=[pl.BlockSpec((r, 256), lambda i: (i, 0)),
                   pl.BlockSpec((r, 128), lambda i: (i, 0)),
                   pl.BlockSpec((r, 128), lambda i: (i, 0))],
        out_shape=[jax.ShapeDtypeStruct((n, 256), F32),
                   jax.ShapeDtypeStruct((n, 128), jnp.int32),
                   jax.ShapeDtypeStruct((n, 128), jnp.int32)],
        compiler_params=pltpu.CompilerParams(
            dimension_semantics=("parallel",)),
    )(x, w1, b1, w2, b2, g, be, ws, wd)


def _encode_edge_body(a_ref, w1, b1, w2, b2, g, be, o_ref):
    t = _silu(jnp.dot(a_ref[...], w1[...], preferred_element_type=F32) + b1[...])
    o = jnp.dot(t.astype(jnp.bfloat16), w2[...],
                preferred_element_type=F32) + b2[...]
    o_ref[...] = _ln(o, g[...], be[...])


def _encode_edge(a, w1, b1, w2, b2, g, be, rows, off):
    de = a.shape[1]
    r = _row_block(math.gcd(rows, off) if off else rows)
    nblk = off // r
    return pl.pallas_call(
        _encode_edge_body,
        grid=(rows // r,),
        in_specs=[
            pl.BlockSpec((r, de), lambda i: (i + nblk, 0)),
            pl.BlockSpec((de, 256), lambda i: (0, 0)),
            _bspec(), _wspec(), _bspec(), _bspec(), _bspec(),
        ],
        out_specs=pl.BlockSpec((r, 256), lambda i: (i, 0)),
        out_shape=jax.ShapeDtypeStruct((rows, 256), F32),
        compiler_params=pltpu.CompilerParams(
            dimension_semantics=("parallel",)),
    )(a, w1, b1, w2, b2, g, be)


def _pack_bf16_pair(o):
    k = o.shape[-1] // 2
    a = lax.bitcast_convert_type(
        o[:, :k].astype(jnp.bfloat16), jnp.uint16).astype(jnp.int32)
    b = lax.bitcast_convert_type(
        o[:, k:].astype(jnp.bfloat16), jnp.uint16).astype(jnp.int32)
    return a | lax.shift_left(b, 16)


def _unpack_bf16_pair(w):
    lo = lax.bitcast_convert_type(
        (w & 0xFFFF).astype(jnp.uint16), jnp.bfloat16).astype(F32)
    hi = lax.bitcast_convert_type(
        lax.shift_right_logical(w, 16).astype(jnp.uint16),
        jnp.bfloat16).astype(F32)
    return jnp.concatenate([lo, hi], axis=-1)


def _edge_update_body(e_ref, gs_ref, gd_ref, w1, b1, w2, b2, g, be, o_ref):
    e = e_ref[...]
    t = jnp.dot(e.astype(jnp.bfloat16), w1[...], preferred_element_type=F32)
    gg = _unpack_bf16_pair(gs_ref[...]) + _unpack_bf16_pair(gd_ref[...])
    t = _silu(t + gg + b1[...])
    o = jnp.dot(t.astype(jnp.bfloat16), w2[...],
                preferred_element_type=F32) + b2[...]
    o_ref[...] = e + _ln(o, g[...], be[...])


def _edge_update(e, gs, gd, w1, b1, w2, b2, g, be):
    n = e.shape[0]
    r = _row_block(n)
    rspec = pl.BlockSpec((r, 256), lambda i: (i, 0))
    gspec = pl.BlockSpec((r, 128), lambda i: (i, 0))
    return pl.pallas_call(
        _edge_update_body,
        grid=(n // r,),
        in_specs=[
            rspec, gspec, gspec,
            _wspec(), _bspec(), _wspec(), _bspec(), _bspec(), _bspec(),
        ],
        out_specs=rspec,
        out_shape=jax.ShapeDtypeStruct((n, 256), F32),
        compiler_params=pltpu.CompilerParams(
            dimension_semantics=("parallel",)),
    )(e, gs, gd, w1, b1, w2, b2, g, be)


def _node_update(h, aggs, w1h, w1a, b1, w2, b2, g, be, ws=None, wd=None):
    n = h.shape[0]
    na = len(aggs)
    r = _row_block(n)
    rspec = pl.BlockSpec((r, 256), lambda i: (i, 0))
    pspec = pl.BlockSpec((r, 128), lambda i: (i, 0))
    params = pltpu.CompilerParams(dimension_semantics=("parallel",))

    def core(refs):
        h_ref = refs[0]
        agg = refs[1][...]
        for a_ref in refs[2:1 + na]:
            agg = agg + a_ref[...]
        w1h_, w1a_, b1_, w2_, b2_, g_, be_ = refs[1 + na:8 + na]
        t = (jnp.dot(h_ref[...], w1h_[...], preferred_element_type=F32)
             + jnp.dot(agg, w1a_[...], preferred_element_type=F32))
        t = _silu(t + b1_[...])
        o = jnp.dot(t, w2_[...], preferred_element_type=F32) + b2_[...]
        return h_ref[...] + _ln(o, g_[...], be_[...])

    if ws is None:
        def body(*refs):
            refs[-1][...] = core(refs)
        return pl.pallas_call(
            body,
            grid=(n // r,),
            in_specs=[rspec] + [rspec] * na + [_wspec(), _wspec(), _bspec(),
                                               _wspec(), _bspec(), _bspec(),
                                               _bspec()],
            out_specs=rspec,
            out_shape=jax.ShapeDtypeStruct((n, 256), F32),
            compiler_params=params,
        )(h, *aggs, w1h, w1a, b1, w2, b2, g, be)

    def body(*refs):
        hnew = core(refs)
        ws_, wd_ = refs[8 + na:10 + na]
        refs[-3][...] = hnew
        refs[-2][...] = _pack_bf16_pair(
            jnp.dot(hnew, ws_[...], preferred_element_type=F32))
        refs[-1][...] = _pack_bf16_pair(
            jnp.dot(hnew, wd_[...], preferred_element_type=F32))

    return pl.pallas_call(
        body,
        grid=(n // r,),
        in_specs=[rspec] + [rspec] * na + [_wspec(), _wspec(), _bspec(),
                                           _wspec(), _bspec(), _bspec(),
                                           _bspec(), _wspec(), _wspec()],
        out_specs=[rspec, pspec, pspec],
        out_shape=[jax.ShapeDtypeStruct((n, 256), F32),
                   jax.ShapeDtypeStruct((n, 128), jnp.int32),
                   jax.ShapeDtypeStruct((n, 128), jnp.int32)],
        compiler_params=params,
    )(h, *aggs, w1h, w1a, b1, w2, b2, g, be, ws, wd)


def _decode_body(h_ref, w1, b1, w2, b2, o_ref):
    t = _silu(jnp.dot(h_ref[...], w1[...], preferred_element_type=F32) + b1[...])
    o_ref[...] = jnp.dot(t, w2[...], preferred_element_type=F32) + b2[...]


def _decode(h, w1, b1, w2, b2):
    n = h.shape[0]
    r = _row_block(n)
    return pl.pallas_call(
        _decode_body,
        grid=(n // r,),
        in_specs=[
            pl.BlockSpec((r, 256), lambda i: (i, 0)),
            _wspec(), _bspec(), _wspec(), _bspec(),
        ],
        out_specs=pl.BlockSpec((r, 256), lambda i: (i, 0)),
        out_shape=jax.ShapeDtypeStruct((n, 256), F32),
        compiler_params=pltpu.CompilerParams(
            dimension_semantics=("parallel",)),
    )(h, w1, b1, w2, b2)


_NC = 2
_NS = 16
_WB = 80


def _chunk(per, cap=200):
    best = 8
    for dcand in range(8, cap + 1, 8):
        if per % dcand == 0:
            best = dcand
    return best


def _edge_split(n_e):
    half = n_e // 2
    lo = max(256, ((half - 10240) // 256) * 256)
    best, best_score = None, -1
    for ea in range(lo, half + 10240 + 1, 256):
        if ea <= 0 or ea >= n_e or (n_e - ea) % 256:
            continue
        score = min(_chunk(ea // 32), _chunk((n_e - ea) // 32))
        if score > best_score:
            best, best_score = ea, score
    return best


def _edge_parts(n_e):
    if n_e % 6400 == 0:
        u = n_e // 6400
        a = max(1, (u * 2) // 5)
        parts = [a, a, u - 2 * a]
        return [p * 6400 for p in parts if p > 0]
    ea = _edge_split(n_e)
    return [ea, n_e - ea]


def _sc_gather(hs, hd, src, dst):
    n_e = src.shape[0]
    dw = hs.shape[1]
    per_w = n_e // (_NC * _NS)
    gk = _chunk(per_w)
    nchunk = per_w // gk
    mesh = plsc.VectorSubcoreMesh(core_axis_name="c", subcore_axis_name="s",
                                  num_cores=_NC, num_subcores=_NS)

    @functools.partial(
        pl.kernel,
        out_type=(jax.ShapeDtypeStruct((n_e, dw), jnp.int32),
                  jax.ShapeDtypeStruct((n_e, dw), jnp.int32)),
        mesh=mesh,
        scratch_types=[
            [pltpu.VMEM((gk,), jnp.int32)] * 2,
            [pltpu.VMEM((gk,), jnp.int32)] * 2,
            [pltpu.VMEM((gk, dw), jnp.int32)] * 2,
            [pltpu.VMEM((gk, dw), jnp.int32)] * 2,
            [pltpu.SemaphoreType.DMA] * 2,
            [pltpu.SemaphoreType.DMA] * 2,
        ])
    def k(hs_hbm, hd_hbm, src_hbm, dst_hbm, gs_hbm, gd_hbm,
          si, di, bs, bd, ss, sd):
        wid = lax.axis_index("s") * _NC + lax.axis_index("c")
        base = wid * per_w

        def start(j, b):
            off = base + j * gk
            pltpu.sync_copy(src_hbm.at[pl.ds(off, gk)], si[b])
            pltpu.sync_copy(dst_hbm.at[pl.ds(off, gk)], di[b])
            pltpu.async_copy(hs_hbm.at[si[b]], bs[b], ss[b])
            pltpu.async_copy(hd_hbm.at[di[b]], bd[b], sd[b])

        def finish(j, b):
            off = base + j * gk
            pltpu.make_async_copy(hs_hbm.at[si[b]], bs[b], ss[b]).wait()
            pltpu.make_async_copy(hd_hbm.at[di[b]], bd[b], sd[b]).wait()
            pltpu.sync_copy(bs[b], gs_hbm.at[pl.ds(off, gk)])
            pltpu.sync_copy(bd[b], gd_hbm.at[pl.ds(off, gk)])

        start(0, 0)
        if nchunk > 1:
            start(1, 1)

        def body(g, carry):
            j0 = 2 * g
            j1 = j0 + 1
            finish(j0, 0)

            @pl.when(j0 + 2 < nchunk)
            def _():
                start(j0 + 2, 0)

            @pl.when(j1 < nchunk)
            def _():
                finish(j1, 1)

            @pl.when(j1 + 2 < nchunk)
            def _():
                start(j1 + 2, 1)

            return carry

        lax.fori_loop(0, (nchunk + 1) // 2, body, 0)

    return k(hs, hd, src, dst)


def _sc_segsum(e, dst, n):
    n_e, d = e.shape
    dh = d // 2
    per_tile = n_e // _NS
    sk = _chunk(per_tile, cap=80)
    nchunk = per_tile // sk
    npad = _NS * _WB * ((n + _NS * _WB - 1) // (_NS * _WB))
    rows_per_tile = npad // _NS
    nwb = rows_per_tile // _WB
    mesh = plsc.VectorSubcoreMesh(core_axis_name="c", subcore_axis_name="s",
                                  num_cores=_NC, num_subcores=_NS)

    @functools.partial(
        pl.kernel,
        out_type=jax.ShapeDtypeStruct((npad, d), F32),
        mesh=mesh,
        scratch_types=[
            [pltpu.VMEM((sk,), jnp.int32)] * 2,
            [pltpu.VMEM((sk, dh), F32)] * 2,
            pltpu.VMEM((_WB, dh), F32),
            pltpu.VMEM_SHARED((npad, dh), F32),
            [pltpu.SemaphoreType.DMA] * 2,
            [pltpu.SemaphoreType.DMA] * 2,
        ])
    def k(e_hbm, dst_hbm, agg_hbm, idxb, ebuf, wbuf, table, six, sro):
        c = lax.axis_index("c")
        s = lax.axis_index("s")
        col0 = c * dh

        zero16 = jnp.zeros((16,), F32)

        def zrow(r, carry):
            for jj in range(dh // 16):
                wbuf[r, pl.ds(jj * 16, 16)] = zero16
            return carry

        lax.fori_loop(0, _WB, zrow, 0)

        def ztab(t, carry):
            pltpu.sync_copy(
                wbuf, table.at[pl.ds(s * rows_per_tile + t * _WB, _WB)])
            return carry

        lax.fori_loop(0, nwb, ztab, 0)
        plsc.subcore_barrier()

        def start(j, b):
            off = s * per_tile + j * sk
            pltpu.async_copy(dst_hbm.at[pl.ds(off, sk)], idxb[b], six[b])
            pltpu.async_copy(e_hbm.at[pl.ds(off, sk), pl.ds(col0, dh)],
                             ebuf[b], sro[b])

        def finish(j, b):
            off = s * per_tile + j * sk
            pltpu.make_async_copy(
                dst_hbm.at[pl.ds(off, sk)], idxb[b], six[b]).wait()
            pltpu.make_async_copy(
                e_hbm.at[pl.ds(off, sk), pl.ds(col0, dh)],
                ebuf[b], sro[b]).wait()
            pltpu.sync_copy(ebuf[b], table.at[idxb[b]], add=True)

        start(0, 0)
        if nchunk > 1:
            start(1, 1)

        def body(g, carry):
            j0 = 2 * g
            j1 = j0 + 1
            finish(j0, 0)

            @pl.when(j0 + 2 < nchunk)
            def _():
                start(j0 + 2, 0)

            @pl.when(j1 < nchunk)
            def _():
                finish(j1, 1)

            @pl.when(j1 + 2 < nchunk)
            def _():
                start(j1 + 2, 1)

            return carry

        lax.fori_loop(0, (nchunk + 1) // 2, body, 0)
        plsc.subcore_barrier()

        def wb(t, carry):
            r0 = s * rows_per_tile + t * _WB
            pltpu.sync_copy(table.at[pl.ds(r0, _WB)], wbuf)
            pltpu.sync_copy(wbuf, agg_hbm.at[pl.ds(r0, _WB),
                                             pl.ds(col0, dh)])
            return carry

        lax.fori_loop(0, nwb, wb, 0)

    return k(e, dst)


def kernel(x, edge_index, edge_attr,
           ne_W1, ne_b1, ne_W2, ne_b2, ne_g, ne_be,
           ee_W1, ee_b1, ee_W2, ee_b2, ee_g, ee_be,
           pe_W1, pe_b1, pe_W2, pe_b2, pe_g, pe_be,
           pn_W1, pn_b1, pn_W2, pn_b2, pn_g, pn_be,
           de_W1, de_b1, de_W2, de_b2):
    n, d = x.shape
    num_layers = pe_W1.shape[0]
    src = edge_index[0]
    dst = edge_index[1]

    n_e = src.shape[0]
    r1 = lambda b: b.reshape(1, -1)
    bf16 = jnp.bfloat16

    rows = _edge_parts(n_e)
    np_ = len(rows)
    offs = [sum(rows[:p]) for p in range(np_)]
    src_h = [src[offs[p]:offs[p] + rows[p]] for p in range(np_)]
    dst_h = [dst[offs[p]:offs[p] + rows[p]] for p in range(np_)]

    h, hs, hd = _encode_node(x, ne_W1, r1(ne_b1), ne_W2, r1(ne_b2),
                             r1(ne_g), r1(ne_be),
                             pe_W1[0][d:2 * d], pe_W1[0][2 * d:])
    e = [
        _encode_edge(edge_attr, ee_W1, r1(ee_b1), ee_W2.astype(bf16),
                     r1(ee_b2), r1(ee_g), r1(ee_be), rows[p], offs[p])
        for p in range(np_)
    ]

    for i in range(num_layers):
        agg = [None] * np_
        for p in range(np_):
            gs, gd = _sc_gather(hs, hd, src_h[p], dst_h[p])
            e[p] = _edge_update(e[p], gs, gd, pe_W1[i][:d].astype(bf16),
                                r1(pe_b1[i]), pe_W2[i].astype(bf16),
                                r1(pe_b2[i]), r1(pe_g[i]), r1(pe_be[i]))
            agg[p] = _sc_segsum(e[p], dst_h[p], n)
        if i + 1 < num_layers:
            h, hs, hd = _node_update(
                h, agg, pn_W1[i][:d], pn_W1[i][d:], r1(pn_b1[i]),
                pn_W2[i], r1(pn_b2[i]), r1(pn_g[i]), r1(pn_be[i]),
                pe_W1[i + 1][d:2 * d], pe_W1[i + 1][2 * d:])
        else:
            h = _node_update(
                h, agg, pn_W1[i][:d], pn_W1[i][d:], r1(pn_b1[i]),
                pn_W2[i], r1(pn_b2[i]), r1(pn_g[i]), r1(pn_be[i]))

    return _decode(h, de_W1, r1(de_b1), de_W2, r1(de_b2))
